# baseline (speedup 1.0000x reference)
import jax
import jax.numpy as jnp
from jax.experimental import pallas as pl
from jax.experimental.pallas import tpu as pltpu

_N_CHUNKS = 8


def _dma_copy(x_hbm, o_hbm, sems):
    n = x_hbm.shape[0]
    chunk = n // _N_CHUNKS
    for i in range(_N_CHUNKS):
        sl = pl.ds(i * chunk, chunk)
        pltpu.make_async_copy(x_hbm.at[sl], o_hbm.at[sl], sems.at[i]).start()
    for i in range(_N_CHUNKS):
        sl = pl.ds(i * chunk, chunk)
        pltpu.make_async_copy(x_hbm.at[sl], o_hbm.at[sl], sems.at[i]).wait()


def kernel(trainable, batch_vec):
    n, d = trainable.shape
    return pl.pallas_call(
        _dma_copy,
        in_specs=[pl.BlockSpec(memory_space=pl.ANY)],
        out_specs=pl.BlockSpec(memory_space=pl.ANY),
        out_shape=jax.ShapeDtypeStruct((n, d), trainable.dtype),
        scratch_shapes=[pltpu.SemaphoreType.DMA((_N_CHUNKS,))],
    )(trainable)


# speedup vs baseline: 26.0682x; 26.0682x over previous
import functools
import jax
import jax.numpy as jnp
from jax import lax
from jax.experimental import pallas as pl
from jax.experimental.pallas import tpu as pltpu, tpu_sc as plsc

_CHUNK = 400


def kernel(trainable, batch_vec):
    n, d = trainable.shape
    n_chunks = pl.cdiv(n, _CHUNK)
    info = plsc.get_sparse_core_info()
    nc, ns = info.num_cores, info.num_subcores
    nw = nc * ns
    max_per_w = -(-n_chunks // nw)
    mesh = plsc.VectorSubcoreMesh(core_axis_name="c", subcore_axis_name="s")

    @functools.partial(
        pl.kernel,
        mesh=mesh,
        out_type=jax.ShapeDtypeStruct((n, d), jnp.float32),
        scratch_types=[pltpu.VMEM((_CHUNK, d), jnp.float32)],
    )
    def sc_copy(x_hbm, o_hbm, buf):
        wid = lax.axis_index("s") * nc + lax.axis_index("c")
        for i in range(max_per_w):
            cid = wid + nw * i

            @pl.when(cid < n_chunks)
            def _():
                base = cid * _CHUNK
                pltpu.sync_copy(x_hbm.at[pl.ds(base, _CHUNK)], buf)
                pltpu.sync_copy(buf, o_hbm.at[pl.ds(base, _CHUNK)])

    return sc_copy(trainable)


# speedup vs baseline: 49.3326x; 1.8924x over previous
import jax
import jax.numpy as jnp
from jax.experimental import pallas as pl
from jax.experimental.pallas import tpu as pltpu

_ROWS_PER_BLOCK = 25000


def _copy_block(x_ref, o_ref):
    o_ref[...] = x_ref[...]


def kernel(trainable, batch_vec):
    n, d = trainable.shape
    grid = pl.cdiv(n, _ROWS_PER_BLOCK)
    return pl.pallas_call(
        _copy_block,
        grid=(grid,),
        in_specs=[pl.BlockSpec((_ROWS_PER_BLOCK, d), lambda i: (i, 0))],
        out_specs=pl.BlockSpec((_ROWS_PER_BLOCK, d), lambda i: (i, 0)),
        out_shape=jax.ShapeDtypeStruct((n, d), trainable.dtype),
        compiler_params=pltpu.CompilerParams(
            dimension_semantics=("parallel",)
        ),
    )(trainable)
